# baseline (speedup 1.0000x reference)
import functools

import jax
import jax.numpy as jnp
from jax import lax
from jax.experimental import pallas as pl
from jax.experimental.pallas import tpu as pltpu
from jax.experimental.pallas import tpu_sc as plsc

N = 10000
NP = 10008
NG = 64
E = 320000
CH = 128
NC, NS = 2, 16
KF = 78
KCH = 80
EROWS = E // CH
F = 32


def _mesh():
    return plsc.VectorSubcoreMesh(
        core_axis_name="c", subcore_axis_name="s",
        num_cores=NC, num_subcores=NS)


_SC_PARAMS = pltpu.CompilerParams(use_tc_tiling_on_sc=False)


def _load_idx(e_hbm, fill_hbm, idx, wid):
    pltpu.sync_copy(e_hbm.at[pl.ds(wid * KF, KF)], idx.at[pl.ds(0, KF)])
    pltpu.sync_copy(e_hbm.at[KF * NS * NC + wid // 8, pl.ds((wid % 8) * 16, 16)],
                    idx.at[KF, pl.ds(0, 16)])
    pltpu.sync_copy(fill_hbm.at[pl.ds(0, 112)], idx.at[KF, pl.ds(16, 112)])
    pltpu.sync_copy(fill_hbm, idx.at[KF + 1])


@functools.partial(
    pl.kernel,
    out_type=jax.ShapeDtypeStruct((NC, NP), jnp.float32),
    mesh=_mesh(),
    compiler_params=_SC_PARAMS,
    scratch_types=[
        pltpu.VMEM((KCH, CH), jnp.int32),
        pltpu.VMEM((CH,), jnp.float32),
        pltpu.VMEM_SHARED((NP,), jnp.float32),
    ],
)
def _sc_degree(dst_hbm, ones_hbm, zeros_hbm, fill_hbm, out_hbm,
               idx_d, ones_v, deg_sh):
    cid = lax.axis_index("c")
    sid = lax.axis_index("s")
    wid = cid * NS + sid
    pltpu.sync_copy(ones_hbm, ones_v)
    _load_idx(dst_hbm, fill_hbm, idx_d, wid)

    @pl.when(sid == 0)
    def _():
        pltpu.sync_copy(zeros_hbm, deg_sh)

    plsc.subcore_barrier()

    def body(j, carry):
        pltpu.sync_copy(ones_v, deg_sh.at[idx_d.at[j]], add=True)
        return carry

    lax.fori_loop(0, KCH, body, 0)
    plsc.subcore_barrier()

    @pl.when(sid == 0)
    def _():
        pltpu.sync_copy(deg_sh, out_hbm.at[cid])


@functools.partial(
    pl.kernel,
    out_type=jax.ShapeDtypeStruct((NC, NP, F), jnp.float32),
    mesh=_mesh(),
    compiler_params=_SC_PARAMS,
    scratch_types=[
        pltpu.VMEM((KCH, CH), jnp.int32),
        pltpu.VMEM((KCH, CH), jnp.int32),
        pltpu.VMEM((CH, F), jnp.float32),
        pltpu.VMEM((CH, F), jnp.float32),
        pltpu.VMEM((CH, F), jnp.float32),
        pltpu.VMEM((CH, F), jnp.float32),
        pltpu.VMEM_SHARED((NP, F), jnp.float32),
        pltpu.VMEM_SHARED((NP, F), jnp.float32),
        pltpu.SemaphoreType.DMA,
        pltpu.SemaphoreType.DMA,
        pltpu.SemaphoreType.DMA,
        pltpu.SemaphoreType.DMA,
        pltpu.SemaphoreType.DMA,
        pltpu.SemaphoreType.DMA,
        pltpu.SemaphoreType.DMA,
        pltpu.SemaphoreType.DMA,
    ],
)
def _sc_agg(y_hbm, src_hbm, dst_hbm, zeros_hbm, fill_hbm, out_hbm,
            idx_s, idx_d, rows0, rows1, rows2, rows3, agg_sh, y_sh,
            sem0, sem1, sem2, sem3, ssem0, ssem1, ssem2, ssem3):
    cid = lax.axis_index("c")
    sid = lax.axis_index("s")
    wid = cid * NS + sid
    bufs = (rows0, rows1, rows2, rows3)
    sems = (sem0, sem1, sem2, sem3)
    ssems = (ssem0, ssem1, ssem2, ssem3)
    _load_idx(src_hbm, fill_hbm, idx_s, wid)
    _load_idx(dst_hbm, fill_hbm, idx_d, wid)

    zc = 1248
    acc_src = zeros_hbm

    @pl.when((sid < 8) & (cid == 0))
    def _():
        pltpu.sync_copy(y_hbm.at[pl.ds(sid * zc, zc)],
                        agg_sh.at[pl.ds(sid * zc, zc)])

    @pl.when((sid == 0) & (cid == 0))
    def _():
        pltpu.sync_copy(y_hbm.at[pl.ds(8 * zc, NP - 8 * zc)],
                        agg_sh.at[pl.ds(8 * zc, NP - 8 * zc)])

    @pl.when((sid < 8) & (cid == 1))
    def _():
        pltpu.sync_copy(acc_src.at[pl.ds(sid * zc, zc)],
                        agg_sh.at[pl.ds(sid * zc, zc)])

    @pl.when((sid == 0) & (cid == 1))
    def _():
        pltpu.sync_copy(acc_src.at[pl.ds(8 * zc, NP - 8 * zc)],
                        agg_sh.at[pl.ds(8 * zc, NP - 8 * zc)])

    @pl.when(sid >= 8)
    def _():
        t = sid - 8
        pltpu.sync_copy(y_hbm.at[pl.ds(t * zc, zc)],
                        y_sh.at[pl.ds(t * zc, zc)])

    @pl.when(sid == 8)
    def _():
        pltpu.sync_copy(y_hbm.at[pl.ds(8 * zc, NP - 8 * zc)],
                        y_sh.at[pl.ds(8 * zc, NP - 8 * zc)])

    plsc.subcore_barrier()

    for b in range(3):
        pltpu.async_copy(y_sh.at[idx_s.at[b]], bufs[b], sems[b])

    def body(i, carry):
        base = 4 * i
        for b in range(4):
            c = base + b
            nb = (b + 3) % 4
            pltpu.make_async_copy(y_sh.at[idx_s.at[c]], bufs[b],
                                  sems[b]).wait()
            pltpu.async_copy(bufs[b], agg_sh.at[idx_d.at[c]], ssems[b],
                             add=True)

            @pl.when(c + 3 < KCH)
            def _():
                @pl.when(c > 0)
                def _():
                    pltpu.make_async_copy(
                        bufs[nb], agg_sh.at[idx_d.at[c - 1]],
                        ssems[nb]).wait()

                pltpu.async_copy(y_sh.at[idx_s.at[c + 3]], bufs[nb],
                                 sems[nb])
        return carry

    lax.fori_loop(0, KCH // 4, body, 0)
    for b in range(4):
        pltpu.make_async_copy(bufs[b], agg_sh.at[idx_d.at[KCH - 4 + b]],
                              ssems[b]).wait()
    plsc.subcore_barrier()

    @pl.when(sid < 8)
    def _():
        pltpu.sync_copy(agg_sh.at[pl.ds(sid * zc, zc)],
                        out_hbm.at[cid, pl.ds(sid * zc, zc)])

    @pl.when(sid == 8)
    def _():
        pltpu.sync_copy(agg_sh.at[pl.ds(8 * zc, NP - 8 * zc)],
                        out_hbm.at[cid, pl.ds(8 * zc, NP - 8 * zc)])


def _tc_layer1_body(x_ref, w_ref, degp_ref, y_ref, dinv_ref):
    deg = degp_ref[0, :] + degp_ref[1, :] + 1.0
    dinv = lax.rsqrt(deg).reshape(NP, 1)
    xw = jnp.dot(x_ref[...], w_ref[...], preferred_element_type=jnp.float32)
    y_ref[:N, :] = xw * dinv[:N, :]
    y_ref[N:, :] = jnp.zeros((NP - N, F), jnp.float32)
    dinv_ref[...] = dinv


_tc_layer1 = pl.pallas_call(
    _tc_layer1_body,
    out_shape=(jax.ShapeDtypeStruct((NP, F), jnp.float32),
               jax.ShapeDtypeStruct((NP, 1), jnp.float32)),
)


def _tc_combine_body(aggp_ref, dinv_ref, b_ref, w_ref, out_ref):
    dinv = dinv_ref[...]
    h = jnp.maximum((aggp_ref[0] + aggp_ref[1]) * dinv + b_ref[...], 0.0)
    hw = jnp.dot(h, w_ref[...], preferred_element_type=jnp.float32)
    row = lax.broadcasted_iota(jnp.int32, (NP, 1), 0)
    out_ref[...] = jnp.where(row < N, hw * dinv, 0.0)


_tc_combine = pl.pallas_call(
    _tc_combine_body,
    out_shape=jax.ShapeDtypeStruct((NP, F), jnp.float32),
)


def _tc_final_body(aggp_ref, dinv_ref, b_ref, bt_ref,
                   wfc_ref, bfc_ref, out_ref):
    agg = aggp_ref[0, :N, :] + aggp_ref[1, :N, :]
    h3 = agg * dinv_ref[:N, :] + b_ref[...]
    gid = lax.broadcasted_iota(jnp.int32, (NG, N), 0)
    oh = (bt_ref[...] == gid).astype(jnp.float32)
    sums = jnp.dot(oh, h3, preferred_element_type=jnp.float32)
    counts = jnp.dot(oh, jnp.ones((N, 1), jnp.float32),
                     preferred_element_type=jnp.float32)
    pooled = sums / jnp.maximum(counts, 1.0)
    out_ref[...] = jnp.dot(pooled, wfc_ref[...],
                           preferred_element_type=jnp.float32) + bfc_ref[...]


_tc_final = pl.pallas_call(
    _tc_final_body,
    out_shape=jax.ShapeDtypeStruct((NG, 10), jnp.float32),
)


def kernel(x, edge_index, batch, W1, b1, W2, b2, W3, b3, Wfc, bfc):
    f32 = jnp.float32
    ei = edge_index.astype(jnp.int32)
    src2 = ei[0].reshape(EROWS, CH)
    dst2 = ei[1].reshape(EROWS, CH)
    fill_c = jnp.full((CH,), N, jnp.int32)
    bt = batch.astype(jnp.int32).reshape(1, N)
    zeros2 = jnp.zeros((NP, F), f32)
    zeros1 = jnp.zeros((NP,), f32)
    ones_c = jnp.ones((CH,), f32)

    degp = _sc_degree(dst2, ones_c, zeros1, fill_c)
    y1, dinv = _tc_layer1(x, W1, degp)
    a1 = _sc_agg(y1, src2, dst2, zeros2, fill_c)
    y2 = _tc_combine(a1, dinv, b1.reshape(1, F), W2)
    a2 = _sc_agg(y2, src2, dst2, zeros2, fill_c)
    y3 = _tc_combine(a2, dinv, b2.reshape(1, F), W3)
    a3 = _sc_agg(y3, src2, dst2, zeros2, fill_c)
    out = _tc_final(a3, dinv, b3.reshape(1, F), bt, Wfc, bfc.reshape(1, 10))
    return out


# speedup vs baseline: 1.0656x; 1.0656x over previous
import functools

import jax
import jax.numpy as jnp
from jax import lax
from jax.experimental import pallas as pl
from jax.experimental.pallas import tpu as pltpu
from jax.experimental.pallas import tpu_sc as plsc

N = 10000
NP = 10008
NG = 64
E = 320000
CH = 128
NC, NS = 2, 16
KCH = 80
EP = NC * NS * KCH * CH
EROWS = EP // CH
F = 32


def _mesh():
    return plsc.VectorSubcoreMesh(
        core_axis_name="c", subcore_axis_name="s",
        num_cores=NC, num_subcores=NS)


_SC_PARAMS = pltpu.CompilerParams(use_tc_tiling_on_sc=False)


@functools.partial(
    pl.kernel,
    out_type=jax.ShapeDtypeStruct((NC, NP), jnp.float32),
    mesh=_mesh(),
    compiler_params=_SC_PARAMS,
    scratch_types=[
        pltpu.VMEM((KCH, CH), jnp.int32),
        pltpu.VMEM((CH,), jnp.float32),
        pltpu.VMEM_SHARED((NP,), jnp.float32),
    ],
)
def _sc_degree(dst_hbm, ones_hbm, zeros_hbm, out_hbm, idx_d, ones_v, deg_sh):
    cid = lax.axis_index("c")
    sid = lax.axis_index("s")
    wid = cid * NS + sid
    pltpu.sync_copy(ones_hbm, ones_v)
    pltpu.sync_copy(dst_hbm.at[pl.ds(wid * KCH, KCH)], idx_d)

    @pl.when(sid == 0)
    def _():
        pltpu.sync_copy(zeros_hbm, deg_sh)

    plsc.subcore_barrier()

    def body(j, carry):
        pltpu.sync_copy(ones_v, deg_sh.at[idx_d.at[j]], add=True)
        return carry

    lax.fori_loop(0, KCH, body, 0)
    plsc.subcore_barrier()

    @pl.when(sid == 0)
    def _():
        pltpu.sync_copy(deg_sh, out_hbm.at[cid])


@functools.partial(
    pl.kernel,
    out_type=jax.ShapeDtypeStruct((NC, NP, F), jnp.float32),
    mesh=_mesh(),
    compiler_params=_SC_PARAMS,
    scratch_types=[
        pltpu.VMEM((KCH, CH), jnp.int32),
        pltpu.VMEM((KCH, CH), jnp.int32),
        pltpu.VMEM((CH, F), jnp.float32),
        pltpu.VMEM((CH, F), jnp.float32),
        pltpu.VMEM((CH, F), jnp.float32),
        pltpu.VMEM((CH, F), jnp.float32),
        pltpu.VMEM_SHARED((NP, F), jnp.float32),
        pltpu.VMEM_SHARED((NP, F), jnp.float32),
        pltpu.SemaphoreType.DMA,
        pltpu.SemaphoreType.DMA,
        pltpu.SemaphoreType.DMA,
        pltpu.SemaphoreType.DMA,
        pltpu.SemaphoreType.DMA,
        pltpu.SemaphoreType.DMA,
        pltpu.SemaphoreType.DMA,
        pltpu.SemaphoreType.DMA,
    ],
)
def _sc_agg(y_hbm, src_hbm, dst_hbm, zeros_hbm, out_hbm,
            idx_s, idx_d, rows0, rows1, rows2, rows3, agg_sh, y_sh,
            sem0, sem1, sem2, sem3, ssem0, ssem1, ssem2, ssem3):
    cid = lax.axis_index("c")
    sid = lax.axis_index("s")
    wid = cid * NS + sid
    bufs = (rows0, rows1, rows2, rows3)
    sems = (sem0, sem1, sem2, sem3)
    ssems = (ssem0, ssem1, ssem2, ssem3)
    pltpu.sync_copy(src_hbm.at[pl.ds(wid * KCH, KCH)], idx_s)
    pltpu.sync_copy(dst_hbm.at[pl.ds(wid * KCH, KCH)], idx_d)

    zc = 1248
    acc_src = zeros_hbm

    @pl.when((sid < 8) & (cid == 0))
    def _():
        pltpu.sync_copy(y_hbm.at[pl.ds(sid * zc, zc)],
                        agg_sh.at[pl.ds(sid * zc, zc)])

    @pl.when((sid == 0) & (cid == 0))
    def _():
        pltpu.sync_copy(y_hbm.at[pl.ds(8 * zc, NP - 8 * zc)],
                        agg_sh.at[pl.ds(8 * zc, NP - 8 * zc)])

    @pl.when((sid < 8) & (cid == 1))
    def _():
        pltpu.sync_copy(acc_src.at[pl.ds(sid * zc, zc)],
                        agg_sh.at[pl.ds(sid * zc, zc)])

    @pl.when((sid == 0) & (cid == 1))
    def _():
        pltpu.sync_copy(acc_src.at[pl.ds(8 * zc, NP - 8 * zc)],
                        agg_sh.at[pl.ds(8 * zc, NP - 8 * zc)])

    @pl.when(sid >= 8)
    def _():
        t = sid - 8
        pltpu.sync_copy(y_hbm.at[pl.ds(t * zc, zc)],
                        y_sh.at[pl.ds(t * zc, zc)])

    @pl.when(sid == 8)
    def _():
        pltpu.sync_copy(y_hbm.at[pl.ds(8 * zc, NP - 8 * zc)],
                        y_sh.at[pl.ds(8 * zc, NP - 8 * zc)])

    plsc.subcore_barrier()

    for b in range(3):
        pltpu.async_copy(y_sh.at[idx_s.at[b]], bufs[b], sems[b])

    def body(i, carry):
        base = 4 * i
        for b in range(4):
            c = base + b
            nb = (b + 3) % 4
            pltpu.make_async_copy(y_sh.at[idx_s.at[c]], bufs[b],
                                  sems[b]).wait()
            pltpu.async_copy(bufs[b], agg_sh.at[idx_d.at[c]], ssems[b],
                             add=True)

            @pl.when(c + 3 < KCH)
            def _():
                @pl.when(c > 0)
                def _():
                    pltpu.make_async_copy(
                        bufs[nb], agg_sh.at[idx_d.at[c - 1]],
                        ssems[nb]).wait()

                pltpu.async_copy(y_sh.at[idx_s.at[c + 3]], bufs[nb],
                                 sems[nb])
        return carry

    lax.fori_loop(0, KCH // 4, body, 0)
    for b in range(4):
        pltpu.make_async_copy(bufs[b], agg_sh.at[idx_d.at[KCH - 4 + b]],
                              ssems[b]).wait()
    plsc.subcore_barrier()

    @pl.when(sid < 8)
    def _():
        pltpu.sync_copy(agg_sh.at[pl.ds(sid * zc, zc)],
                        out_hbm.at[cid, pl.ds(sid * zc, zc)])

    @pl.when(sid == 8)
    def _():
        pltpu.sync_copy(agg_sh.at[pl.ds(8 * zc, NP - 8 * zc)],
                        out_hbm.at[cid, pl.ds(8 * zc, NP - 8 * zc)])


def _tc_layer1_body(x_ref, w_ref, degp_ref, y_ref, dinv_ref):
    deg = degp_ref[0, :] + degp_ref[1, :] + 1.0
    dinv = lax.rsqrt(deg).reshape(NP, 1)
    xw = jnp.dot(x_ref[...], w_ref[...], preferred_element_type=jnp.float32)
    y_ref[:N, :] = xw * dinv[:N, :]
    y_ref[N:, :] = jnp.zeros((NP - N, F), jnp.float32)
    dinv_ref[...] = dinv


_tc_layer1 = pl.pallas_call(
    _tc_layer1_body,
    out_shape=(jax.ShapeDtypeStruct((NP, F), jnp.float32),
               jax.ShapeDtypeStruct((NP, 1), jnp.float32)),
)


def _tc_combine_body(aggp_ref, dinv_ref, b_ref, w_ref, out_ref):
    dinv = dinv_ref[...]
    h = jnp.maximum((aggp_ref[0] + aggp_ref[1]) * dinv + b_ref[...], 0.0)
    hw = jnp.dot(h, w_ref[...], preferred_element_type=jnp.float32)
    row = lax.broadcasted_iota(jnp.int32, (NP, 1), 0)
    out_ref[...] = jnp.where(row < N, hw * dinv, 0.0)


_tc_combine = pl.pallas_call(
    _tc_combine_body,
    out_shape=jax.ShapeDtypeStruct((NP, F), jnp.float32),
)


def _tc_final_body(aggp_ref, dinv_ref, b_ref, bt_ref,
                   wfc_ref, bfc_ref, out_ref):
    agg = aggp_ref[0, :N, :] + aggp_ref[1, :N, :]
    h3 = agg * dinv_ref[:N, :] + b_ref[...]
    gid = lax.broadcasted_iota(jnp.int32, (NG, N), 0)
    oh = (bt_ref[...] == gid).astype(jnp.float32)
    sums = jnp.dot(oh, h3, preferred_element_type=jnp.float32)
    counts = jnp.dot(oh, jnp.ones((N, 1), jnp.float32),
                     preferred_element_type=jnp.float32)
    pooled = sums / jnp.maximum(counts, 1.0)
    out_ref[...] = jnp.dot(pooled, wfc_ref[...],
                           preferred_element_type=jnp.float32) + bfc_ref[...]


_tc_final = pl.pallas_call(
    _tc_final_body,
    out_shape=jax.ShapeDtypeStruct((NG, 10), jnp.float32),
)


def kernel(x, edge_index, batch, W1, b1, W2, b2, W3, b3, Wfc, bfc):
    f32 = jnp.float32
    ei = edge_index.astype(jnp.int32)
    pad = jnp.full((EP - E,), N, jnp.int32)
    src2 = jnp.concatenate([ei[0], pad]).reshape(EROWS, CH)
    dst2 = jnp.concatenate([ei[1], pad]).reshape(EROWS, CH)
    bt = batch.astype(jnp.int32).reshape(1, N)
    zeros2 = jnp.zeros((NP, F), f32)
    zeros1 = jnp.zeros((NP,), f32)
    ones_c = jnp.ones((CH,), f32)

    degp = _sc_degree(dst2, ones_c, zeros1)
    y1, dinv = _tc_layer1(x, W1, degp)
    a1 = _sc_agg(y1, src2, dst2, zeros2)
    y2 = _tc_combine(a1, dinv, b1.reshape(1, F), W2)
    a2 = _sc_agg(y2, src2, dst2, zeros2)
    y3 = _tc_combine(a2, dinv, b2.reshape(1, F), W3)
    a3 = _sc_agg(y3, src2, dst2, zeros2)
    out = _tc_final(a3, dinv, b3.reshape(1, F), bt, Wfc, bfc.reshape(1, 10))
    return out
